# Initial kernel scaffold; baseline (speedup 1.0000x reference)
import jax
import jax.numpy as jnp
from jax.experimental import pallas as pl


def kernel(inputs, edge_index, W1, b1, W2, b2):
    raise NotImplementedError("write your pallas kernel here")


# speedup vs baseline: 7.1237x; 7.1237x over previous
import functools

import jax
import jax.numpy as jnp
from jax import lax
from jax.experimental import pallas as pl
from jax.experimental.pallas import tpu as pltpu
from jax.experimental.pallas import tpu_sc as plsc

N = 10000
E = 320000
F = 128
H = 256
O = 2

NC = 2
NS = 16
NW = NC * NS
L = 16
NP = 10240
RPT = NP // NS
EPW = E // NW
C3 = 128
K3 = 79
E3 = NW * K3 * C3
NPF = 2 * NP


def _mesh():
    return plsc.VectorSubcoreMesh(core_axis_name="c", subcore_axis_name="s")


def _sc_degrees(src, dst, zflat):
    @functools.partial(
        pl.kernel,
        mesh=_mesh(),
        compiler_params=pltpu.CompilerParams(needs_layout_passes=False),
        out_type=jax.ShapeDtypeStruct((NW, NPF), jnp.float32),
        scratch_types=[
            pltpu.VMEM((EPW,), jnp.int32),
            pltpu.VMEM((EPW,), jnp.int32),
            pltpu.VMEM((NPF,), jnp.float32),
        ],
    )
    def k(src_hbm, dst_hbm, zf_hbm, out_hbm, sidx_v, didx_v, cnt_v):
        c = lax.axis_index("c")
        s = lax.axis_index("s")
        wid = s * NC + c
        base = wid * EPW
        pltpu.sync_copy(zf_hbm, cnt_v)
        pltpu.sync_copy(src_hbm.at[pl.ds(base, EPW)], sidx_v)
        pltpu.sync_copy(dst_hbm.at[pl.ds(base, EPW)], didx_v)
        ones = jnp.ones((L,), jnp.float32)
        offs = jnp.full((L,), NP, jnp.int32)

        def body(g, carry):
            i0 = g * L
            plsc.addupdate_scatter(cnt_v, [sidx_v[pl.ds(i0, L)]], ones)
            plsc.addupdate_scatter(cnt_v, [didx_v[pl.ds(i0, L)] + offs], ones)
            return carry

        lax.fori_loop(0, EPW // L, body, 0)
        pltpu.sync_copy(cnt_v, out_hbm.at[wid])

    return k(src, dst, zflat)


def _sc_prop128(xs, srcp, dstp, z2d):
    @functools.partial(
        pl.kernel,
        mesh=_mesh(),
        compiler_params=pltpu.CompilerParams(needs_layout_passes=False),
        out_type=jax.ShapeDtypeStruct((NC, NP, F), jnp.float32),
        scratch_types=[
            pltpu.VMEM((K3, C3), jnp.int32),
            pltpu.VMEM((K3, C3), jnp.int32),
            pltpu.VMEM((C3, F), jnp.float32),
            pltpu.VMEM_SHARED((NP, F), jnp.float32),
            pltpu.SemaphoreType.DMA,
        ],
    )
    def k(xs_hbm, sp_hbm, dp_hbm, z_hbm, out_hbm, src_v, dst_v, rows_v, acc_sh, sem):
        c = lax.axis_index("c")
        s = lax.axis_index("s")
        wid = s * NC + c
        pltpu.sync_copy(z_hbm, acc_sh.at[pl.ds(s * RPT, RPT)])
        pltpu.sync_copy(sp_hbm.at[wid], src_v)
        pltpu.sync_copy(dp_hbm.at[wid], dst_v)
        plsc.subcore_barrier()

        def body(j, carry):
            pltpu.async_copy(xs_hbm.at[src_v.at[j]], rows_v, sem).wait()
            pltpu.sync_copy(rows_v, acc_sh.at[dst_v.at[j]], add=True)
            return carry

        lax.fori_loop(0, K3, body, 0)
        plsc.subcore_barrier()
        pltpu.sync_copy(
            acc_sh.at[pl.ds(s * RPT, RPT)], out_hbm.at[c, pl.ds(s * RPT, RPT)]
        )

    return k(xs, srcp, dstp, z2d)


def _sc_prop2(gflat, src, dst, zflat):
    @functools.partial(
        pl.kernel,
        mesh=_mesh(),
        compiler_params=pltpu.CompilerParams(needs_layout_passes=False),
        out_type=jax.ShapeDtypeStruct((NW, NPF), jnp.float32),
        scratch_types=[
            pltpu.VMEM((NPF,), jnp.float32),
            pltpu.VMEM((NPF,), jnp.float32),
            pltpu.VMEM((EPW,), jnp.int32),
            pltpu.VMEM((EPW,), jnp.int32),
        ],
    )
    def k(g_hbm, src_hbm, dst_hbm, zf_hbm, out_hbm, g_v, acc_v, sidx_v, didx_v):
        c = lax.axis_index("c")
        s = lax.axis_index("s")
        wid = s * NC + c
        base = wid * EPW
        pltpu.sync_copy(zf_hbm, acc_v)
        pltpu.sync_copy(g_hbm, g_v)
        pltpu.sync_copy(src_hbm.at[pl.ds(base, EPW)], sidx_v)
        pltpu.sync_copy(dst_hbm.at[pl.ds(base, EPW)], didx_v)
        ones = jnp.full((L,), 1, jnp.int32)

        def body(g, carry):
            i0 = g * L
            si = sidx_v[pl.ds(i0, L)] * 2
            di = didx_v[pl.ds(i0, L)] * 2
            v0 = plsc.load_gather(g_v, [si])
            v1 = plsc.load_gather(g_v, [si + ones])
            plsc.addupdate_scatter(acc_v, [di], v0)
            plsc.addupdate_scatter(acc_v, [di + ones], v1)
            return carry

        lax.fori_loop(0, EPW // L, body, 0)
        pltpu.sync_copy(acc_v, out_hbm.at[wid])

    return k(gflat, src, dst, zflat)


def _tc_count_norm(cnt32):
    def body(c_ref, o_ref):
        o_ref[...] = lax.rsqrt(jnp.maximum(jnp.sum(c_ref[...], axis=0), 1.0))

    return pl.pallas_call(
        body,
        out_shape=jax.ShapeDtypeStruct((NPF // 128, 128), jnp.float32),
    )(cnt32)


def _tc_scale(x_pad, nsrc_col):
    def body(x_ref, n_ref, o_ref):
        o_ref[...] = x_ref[...] * n_ref[...]

    return pl.pallas_call(
        body,
        out_shape=jax.ShapeDtypeStruct((NP, F), jnp.float32),
    )(x_pad, nsrc_col)


def _tc_mlp(P, ndst_col, nsrc_col, W1, b1r, W2):
    R = 1024
    NB = NP // R

    def body(p_ref, nd_ref, ns_ref, w1_ref, b1_ref, w2_ref, o_ref):
        p = p_ref[0] + p_ref[1]
        m = p * nd_ref[...]
        h = jnp.dot(m, w1_ref[...], preferred_element_type=jnp.float32)
        h = jnp.maximum(h + b1_ref[...], 0.0)
        o_ref[...] = jnp.dot(
            h * ns_ref[...], w2_ref[...], preferred_element_type=jnp.float32
        )

    return pl.pallas_call(
        body,
        grid=(NB,),
        in_specs=[
            pl.BlockSpec((NC, R, F), lambda i: (0, i, 0)),
            pl.BlockSpec((R, 1), lambda i: (i, 0)),
            pl.BlockSpec((R, 1), lambda i: (i, 0)),
            pl.BlockSpec((F, H), lambda i: (0, 0)),
            pl.BlockSpec((1, H), lambda i: (0, 0)),
            pl.BlockSpec((H, O), lambda i: (0, 0)),
        ],
        out_specs=pl.BlockSpec((R, O), lambda i: (i, 0)),
        out_shape=jax.ShapeDtypeStruct((NP, O), jnp.float32),
    )(P, ndst_col, nsrc_col, W1, b1r, W2)


def _tc_sum(q32):
    def body(q_ref, o_ref):
        o_ref[...] = jnp.sum(q_ref[...], axis=0)

    return pl.pallas_call(
        body,
        out_shape=jax.ShapeDtypeStruct((NPF // 128, 128), jnp.float32),
    )(q32)


def _tc_final(q2, ndst_col, b2r):
    def body(q_ref, nd_ref, b_ref, o_ref):
        o_ref[...] = q_ref[...] * nd_ref[...] + b_ref[...]

    return pl.pallas_call(
        body,
        out_shape=jax.ShapeDtypeStruct((NP, O), jnp.float32),
    )(q2, ndst_col, b2r)


def kernel(inputs, edge_index, W1, b1, W2, b2):
    src = edge_index[0].astype(jnp.int32)
    dst = edge_index[1].astype(jnp.int32)
    zflat = jnp.zeros((NPF,), jnp.float32)
    z2d = jnp.zeros((RPT, F), jnp.float32)
    x_pad = jnp.pad(inputs, ((0, NP - N), (0, 0)))

    cnt32 = _sc_degrees(src, dst, zflat)
    norms2d = _tc_count_norm(cnt32.reshape(NW, NPF // 128, 128))
    norms = norms2d.reshape(NPF, 1)
    nsrc_col = norms[:NP]
    ndst_col = norms[NP:]
    xs = _tc_scale(x_pad, nsrc_col)

    pad3 = E3 - E
    srcp = jnp.concatenate([src, jnp.zeros((pad3,), jnp.int32)]).reshape(NW, K3, C3)
    dstp = jnp.concatenate([dst, jnp.full((pad3,), N, jnp.int32)]).reshape(NW, K3, C3)
    P = _sc_prop128(xs, srcp, dstp, z2d)

    g = _tc_mlp(P, ndst_col, nsrc_col, W1, b1.reshape(1, H), W2)
    q32 = _sc_prop2(g.reshape(NPF), src, dst, zflat)
    q2d = _tc_sum(q32.reshape(NW, NPF // 128, 128))
    out = _tc_final(q2d.reshape(NP, O), ndst_col, b2.reshape(1, O))
    return out[:N]
